# Initial kernel scaffold; baseline (speedup 1.0000x reference)
import jax
import jax.numpy as jnp
from jax.experimental import pallas as pl


def kernel(X, emb_table, dense_table, fm_bias, W0, b0, W1, b1, W2, b2, W3, b3):
    raise NotImplementedError("write your pallas kernel here")


# speedup vs baseline: 1.6035x; 1.6035x over previous
import jax
import jax.numpy as jnp
from jax import lax
from jax.experimental import pallas as pl
from jax.experimental.pallas import tpu as pltpu

B = 4096
NF = 26
V = 100001
D = 32
BLK = 512
HID0, HID1, HID2 = 1024, 512, 256


def _mlp_body(h0_ref, dv_ref, fmb_ref, w0_ref, b0_ref, w1_ref, b1_ref,
              w2_ref, b2_ref, w3_ref, b3_ref, out_ref):
    h0 = h0_ref[...]
    one = jnp.sum(dv_ref[...], axis=1)
    ssq = jnp.sum(h0 * h0, axis=1)
    ki = lax.broadcasted_iota(jnp.int32, (NF * D, D), 0)
    di = lax.broadcasted_iota(jnp.int32, (NF * D, D), 1)
    fold = (ki % D == di).astype(jnp.float32)
    s = jnp.dot(h0, fold, preferred_element_type=jnp.float32)
    two = 0.5 * (jnp.sum(s * s, axis=1) - ssq)
    fm = one + two + fmb_ref[...]

    h = h0
    for w_ref, b_ref in ((w0_ref, b0_ref), (w1_ref, b1_ref), (w2_ref, b2_ref)):
        h = jnp.dot(h, w_ref[...], preferred_element_type=jnp.float32)
        h = jnp.maximum(h + b_ref[...][None, :], 0.0)
    deep = jnp.maximum(jnp.sum(h * w3_ref[...], axis=1) + b3_ref[...], 0.0)
    out_ref[...] = jax.nn.sigmoid(fm + deep)


_mlp = pl.pallas_call(
    _mlp_body,
    grid=(B // BLK,),
    in_specs=[
        pl.BlockSpec((BLK, NF * D), lambda i: (i, 0)),
        pl.BlockSpec((BLK, NF), lambda i: (i, 0)),
        pl.BlockSpec((1,), lambda i: (0,)),
        pl.BlockSpec((NF * D, HID0), lambda i: (0, 0)),
        pl.BlockSpec((HID0,), lambda i: (0,)),
        pl.BlockSpec((HID0, HID1), lambda i: (0, 0)),
        pl.BlockSpec((HID1,), lambda i: (0,)),
        pl.BlockSpec((HID1, HID2), lambda i: (0, 0)),
        pl.BlockSpec((HID2,), lambda i: (0,)),
        pl.BlockSpec((1, HID2), lambda i: (0, 0)),
        pl.BlockSpec((1,), lambda i: (0,)),
    ],
    out_specs=pl.BlockSpec((BLK,), lambda i: (i,)),
    out_shape=jax.ShapeDtypeStruct((B,), jnp.float32),
)


def kernel(X, emb_table, dense_table, fm_bias, W0, b0, W1, b1, W2, b2, W3, b3):
    X = X.astype(jnp.int32)
    feat = jnp.arange(NF)[None, :]
    embeds = emb_table[feat, X, :]
    dv = dense_table[feat, X, 0]
    h0 = embeds.reshape(B, NF * D)
    return _mlp(h0, dv, fm_bias, W0, b0, W1, b1, W2, b2, W3.reshape(1, HID2), b3)
